# Initial kernel scaffold; baseline (speedup 1.0000x reference)
import jax
import jax.numpy as jnp
from jax.experimental import pallas as pl


def kernel(x, w1, w2, wf1p, wf2p, bcat):
    raise NotImplementedError("write your pallas kernel here")


# speedup vs baseline: 51.1685x; 51.1685x over previous
import jax
import jax.numpy as jnp
from jax.experimental import pallas as pl
from jax.experimental.pallas import tpu as pltpu

_BT = 64


def _lenet_kernel(xq_ref, w1_ref, w2_ref, wf1_ref, wf2_ref, b_ref, o_ref):
    f32 = jnp.float32
    bf16 = jnp.bfloat16
    BT = o_ref.shape[0]

    def mm(a, b):
        return jax.lax.dot_general(a, b, (((1,), (0,)), ((), ())),
                                   preferred_element_type=f32)

    slabs = [xq_ref[t % 2, (t // 2):(t // 2) + 16].reshape(16 * BT, 28)
             for t in range(6)]
    l1 = jnp.concatenate(slabs, axis=1)
    acc = mm(l1, w1_ref[...])
    a2 = jnp.maximum(acc[:, :256], acc[:, 256:])
    y1 = jnp.maximum(a2[:, :128], a2[:, 128:])
    h1 = jnp.maximum(y1 + b_ref[0:1, :], 0.0).astype(bf16)

    l2 = jnp.concatenate([h1[t * BT:(t + 8) * BT] for t in range(5)], axis=1)
    acc2 = mm(l2, w2_ref[...])
    y2 = jnp.maximum(acc2[:, :128], acc2[:, 128:])
    y2 = jnp.maximum(y2 + b_ref[1:2, :], 0.0)
    y2r = y2.reshape(4, 2, BT, 128)
    m4 = jnp.maximum(y2r[:, 0], y2r[:, 1]).astype(bf16)

    l3 = jnp.concatenate([m4[i] for i in range(4)], axis=1)
    f = jnp.maximum(mm(l3, wf1_ref[...]) + b_ref[2:3, :], 0.0).astype(bf16)
    z = mm(f, wf2_ref[...]) + b_ref[3:4, :]
    zmax = jnp.max(z, axis=-1, keepdims=True)
    lse = jnp.log(jnp.sum(jnp.exp(z - zmax), axis=-1, keepdims=True)) + zmax
    o_ref[...] = z - lse


def _const_spec(shape):
    nd = len(shape)
    return pl.BlockSpec(shape, lambda i, _nd=nd: (0,) * _nd)


def kernel(x, w1, w2, wf1p, wf2p, bcat):
    f32 = jnp.float32
    bf16 = jnp.bfloat16
    B = x.shape[0]
    bt = min(_BT, B)

    wc1k = w1[:, 0:5, 0:10]
    t6 = jnp.arange(6)[:, None, None]
    q2 = jnp.arange(2)[None, :, None]
    kh5 = jnp.arange(5)[None, None, :]
    sel1 = (t6 == kh5 + q2).astype(f32)
    iw28 = jnp.arange(28)[:, None, None, None]
    s2 = jnp.arange(2)[None, :, None, None]
    j12 = jnp.arange(12)[None, None, :, None]
    kw5 = jnp.arange(5)[None, None, None, :]
    sel2 = (iw28 == 2 * j12 + s2 + kw5).astype(f32)
    w1c = jnp.einsum('tqh,isjw,hwc->tiqsjc', sel1, sel2, wc1k)
    w1c = w1c.reshape(6, 28, 2, 2, 120)
    w1c = jnp.pad(w1c, ((0, 0), (0, 0), (0, 0), (0, 0), (0, 8)))
    w1c = w1c.reshape(168, 512).astype(bf16)

    wc2k = w2.reshape(25, 128, 128)[:, :10, :20].reshape(5, 5, 10, 20)
    j1a = jnp.arange(12)[:, None, None, None]
    s2b = jnp.arange(2)[None, :, None, None]
    j2a = jnp.arange(4)[None, None, :, None]
    kw5b = jnp.arange(5)[None, None, None, :]
    sel3 = (j1a == 2 * j2a + s2b + kw5b).astype(f32)
    w2c = jnp.einsum('jszw,hwio->hjiszo', sel3, wc2k)
    w2c = w2c.reshape(5, 120, 2, 80)
    w2c = jnp.pad(w2c, ((0, 0), (0, 8), (0, 0), (0, 48)))
    w2c = w2c.reshape(640, 256).astype(bf16)

    wf1c = wf1p.reshape(4, 4, 128, 128)[:, :, :20, :].reshape(4, 80, 128)
    wf1c = jnp.pad(wf1c, ((0, 0), (0, 48), (0, 0))).reshape(512, 128)
    wf1c = wf1c.astype(bf16)
    wf2c = wf2p.astype(bf16)

    lane = jnp.arange(128)
    b1v = jnp.where(lane < 120, bcat[0][lane % 10], 0.0)
    b2v = jnp.where(lane < 80, bcat[1][lane % 20], 0.0)
    bpack = jnp.stack([b1v, b2v, bcat[2], bcat[3]], axis=0)

    xr = x.reshape(B, 28, 28)
    xq = jnp.stack([xr[:, 0::2, :], xr[:, 1::2, :]], 0)
    xq = jnp.transpose(xq, (0, 2, 1, 3))
    xq = jnp.pad(xq, ((0, 0), (0, 4), (0, 0), (0, 0))).astype(bf16)

    flops = 2 * B * (16 * 168 * 512 + 8 * 640 * 256 + 512 * 128 + 128 * 128)
    bytes_accessed = xq.size * 2 + B * 128 * 4 + 2 * (
        w1c.size + w2c.size + wf1c.size + wf2c.size) + bpack.size * 4

    out = pl.pallas_call(
        _lenet_kernel,
        out_shape=jax.ShapeDtypeStruct((B, 128), f32),
        grid=(B // bt,),
        in_specs=[
            pl.BlockSpec((2, 18, bt, 28), lambda i: (0, 0, i, 0)),
            _const_spec((168, 512)),
            _const_spec((640, 256)),
            _const_spec((512, 128)),
            _const_spec((128, 128)),
            _const_spec((4, 128)),
        ],
        out_specs=pl.BlockSpec((bt, 128), lambda i: (i, 0)),
        compiler_params=pltpu.CompilerParams(
            dimension_semantics=("parallel",),
            vmem_limit_bytes=64 * 1024 * 1024),
        cost_estimate=pl.CostEstimate(flops=flops, transcendentals=B * 128,
                                      bytes_accessed=bytes_accessed),
    )(xq, w1c, w2c, wf1c, wf2c, bpack)
    return out[:, :10]


# speedup vs baseline: 57.6129x; 1.1259x over previous
import jax
import jax.numpy as jnp
from jax.experimental import pallas as pl
from jax.experimental.pallas import tpu as pltpu

_BT = 128


def _lenet_kernel(xq_ref, w1_ref, w2_ref, wf1_ref, wf2_ref, b_ref, o_ref):
    f32 = jnp.float32
    bf16 = jnp.bfloat16
    BT = o_ref.shape[0]

    def mm(a, b):
        return jax.lax.dot_general(a, b, (((1,), (0,)), ((), ())),
                                   preferred_element_type=f32)

    slabs = [xq_ref[t % 2, (t // 2):(t // 2) + 16].reshape(16 * BT, 28)
             for t in range(6)]
    l1 = jnp.concatenate(slabs, axis=1)
    acc = mm(l1, w1_ref[...])
    a2 = jnp.maximum(acc[:, :256], acc[:, 256:])
    y1 = jnp.maximum(a2[:, :128], a2[:, 128:])
    h1 = jnp.maximum(y1 + b_ref[0:1, :], 0.0).astype(bf16)

    l2 = jnp.concatenate([h1[t * BT:(t + 8) * BT] for t in range(5)], axis=1)
    acc2 = mm(l2, w2_ref[...])
    y2 = jnp.maximum(acc2[:, :128], acc2[:, 128:])
    y2 = jnp.maximum(y2 + b_ref[1:2, :], 0.0)
    y2r = y2.reshape(4, 2, BT, 128)
    m4 = jnp.maximum(y2r[:, 0], y2r[:, 1]).astype(bf16)

    l3 = jnp.concatenate([m4[i] for i in range(4)], axis=1)
    f = jnp.maximum(mm(l3, wf1_ref[...]) + b_ref[2:3, :], 0.0).astype(bf16)
    z = mm(f, wf2_ref[...]) + b_ref[3:4, :]
    zmax = jnp.max(z, axis=-1, keepdims=True)
    lse = jnp.log(jnp.sum(jnp.exp(z - zmax), axis=-1, keepdims=True)) + zmax
    o_ref[...] = z - lse


def _const_spec(shape):
    nd = len(shape)
    return pl.BlockSpec(shape, lambda i, _nd=nd: (0,) * _nd)


def kernel(x, w1, w2, wf1p, wf2p, bcat):
    f32 = jnp.float32
    bf16 = jnp.bfloat16
    B = x.shape[0]
    bt = min(_BT, B)

    wc1k = w1[:, 0:5, 0:10]
    t6 = jnp.arange(6)[:, None, None]
    q2 = jnp.arange(2)[None, :, None]
    kh5 = jnp.arange(5)[None, None, :]
    sel1 = (t6 == kh5 + q2).astype(f32)
    iw28 = jnp.arange(28)[:, None, None, None]
    s2 = jnp.arange(2)[None, :, None, None]
    j12 = jnp.arange(12)[None, None, :, None]
    kw5 = jnp.arange(5)[None, None, None, :]
    sel2 = (iw28 == 2 * j12 + s2 + kw5).astype(f32)
    w1c = jnp.einsum('tqh,isjw,hwc->tiqsjc', sel1, sel2, wc1k)
    w1c = w1c.reshape(6, 28, 2, 2, 120)
    w1c = jnp.pad(w1c, ((0, 0), (0, 0), (0, 0), (0, 0), (0, 8)))
    w1c = w1c.reshape(168, 512).astype(bf16)

    wc2k = w2.reshape(25, 128, 128)[:, :10, :20].reshape(5, 5, 10, 20)
    j1a = jnp.arange(12)[:, None, None, None]
    s2b = jnp.arange(2)[None, :, None, None]
    j2a = jnp.arange(4)[None, None, :, None]
    kw5b = jnp.arange(5)[None, None, None, :]
    sel3 = (j1a == 2 * j2a + s2b + kw5b).astype(f32)
    w2c = jnp.einsum('jszw,hwio->hjiszo', sel3, wc2k)
    w2c = w2c.reshape(5, 120, 2, 80)
    w2c = jnp.pad(w2c, ((0, 0), (0, 8), (0, 0), (0, 48)))
    w2c = w2c.reshape(640, 256).astype(bf16)

    wf1c = wf1p.reshape(4, 4, 128, 128)[:, :, :20, :].reshape(4, 80, 128)
    wf1c = jnp.pad(wf1c, ((0, 0), (0, 48), (0, 0))).reshape(512, 128)
    wf1c = wf1c.astype(bf16)
    wf2c = wf2p.astype(bf16)

    lane = jnp.arange(128)
    b1v = jnp.where(lane < 120, bcat[0][lane % 10], 0.0)
    b2v = jnp.where(lane < 80, bcat[1][lane % 20], 0.0)
    bpack = jnp.stack([b1v, b2v, bcat[2], bcat[3]], axis=0)

    xr = x.reshape(B, 28, 28)
    xq = jnp.stack([xr[:, 0::2, :], xr[:, 1::2, :]], 0)
    xq = jnp.transpose(xq, (0, 2, 1, 3))
    xq = jnp.pad(xq, ((0, 0), (0, 4), (0, 0), (0, 0))).astype(bf16)

    flops = 2 * B * (16 * 168 * 512 + 8 * 640 * 256 + 512 * 128 + 128 * 128)
    bytes_accessed = xq.size * 2 + B * 128 * 4 + 2 * (
        w1c.size + w2c.size + wf1c.size + wf2c.size) + bpack.size * 4

    out = pl.pallas_call(
        _lenet_kernel,
        out_shape=jax.ShapeDtypeStruct((B, 128), f32),
        grid=(B // bt,),
        in_specs=[
            pl.BlockSpec((2, 18, bt, 28), lambda i: (0, 0, i, 0)),
            _const_spec((168, 512)),
            _const_spec((640, 256)),
            _const_spec((512, 128)),
            _const_spec((128, 128)),
            _const_spec((4, 128)),
        ],
        out_specs=pl.BlockSpec((bt, 128), lambda i: (i, 0)),
        compiler_params=pltpu.CompilerParams(
            dimension_semantics=("parallel",),
            vmem_limit_bytes=64 * 1024 * 1024),
        cost_estimate=pl.CostEstimate(flops=flops, transcendentals=B * 128,
                                      bytes_accessed=bytes_accessed),
    )(xq, w1c, w2c, wf1c, wf2c, bpack)
    return out[:, :10]


# speedup vs baseline: 61.2827x; 1.0637x over previous
import jax
import jax.numpy as jnp
from jax.experimental import pallas as pl
from jax.experimental.pallas import tpu as pltpu

_BT = 256


def _lenet_kernel(xq_ref, w1_ref, w2_ref, wf1_ref, wf2_ref, b_ref, o_ref):
    f32 = jnp.float32
    bf16 = jnp.bfloat16
    BT = o_ref.shape[0]

    def mm(a, b):
        return jax.lax.dot_general(a, b, (((1,), (0,)), ((), ())),
                                   preferred_element_type=f32)

    slabs = [xq_ref[t % 2, (t // 2):(t // 2) + 16].reshape(16 * BT, 28)
             for t in range(6)]
    l1 = jnp.concatenate(slabs, axis=1)
    acc = mm(l1, w1_ref[...])
    a2 = jnp.maximum(acc[:, :256], acc[:, 256:])
    y1 = jnp.maximum(a2[:, :128], a2[:, 128:])
    h1 = jnp.maximum(y1 + b_ref[0:1, :], 0.0).astype(bf16)

    l2 = jnp.concatenate([h1[t * BT:(t + 8) * BT] for t in range(5)], axis=1)
    acc2 = mm(l2, w2_ref[...])
    y2 = jnp.maximum(acc2[:, :128], acc2[:, 128:])
    y2 = jnp.maximum(y2 + b_ref[1:2, :], 0.0)
    y2r = y2.reshape(4, 2, BT, 128)
    m4 = jnp.maximum(y2r[:, 0], y2r[:, 1]).astype(bf16)

    l3 = jnp.concatenate([m4[i] for i in range(4)], axis=1)
    f = jnp.maximum(mm(l3, wf1_ref[...]) + b_ref[2:3, :], 0.0).astype(bf16)
    z = mm(f, wf2_ref[...]) + b_ref[3:4, :]
    zmax = jnp.max(z, axis=-1, keepdims=True)
    lse = jnp.log(jnp.sum(jnp.exp(z - zmax), axis=-1, keepdims=True)) + zmax
    o_ref[...] = z - lse


def _const_spec(shape):
    nd = len(shape)
    return pl.BlockSpec(shape, lambda i, _nd=nd: (0,) * _nd)


def kernel(x, w1, w2, wf1p, wf2p, bcat):
    f32 = jnp.float32
    bf16 = jnp.bfloat16
    B = x.shape[0]
    bt = min(_BT, B)

    wc1k = w1[:, 0:5, 0:10]
    t6 = jnp.arange(6)[:, None, None]
    q2 = jnp.arange(2)[None, :, None]
    kh5 = jnp.arange(5)[None, None, :]
    sel1 = (t6 == kh5 + q2).astype(f32)
    iw28 = jnp.arange(28)[:, None, None, None]
    s2 = jnp.arange(2)[None, :, None, None]
    j12 = jnp.arange(12)[None, None, :, None]
    kw5 = jnp.arange(5)[None, None, None, :]
    sel2 = (iw28 == 2 * j12 + s2 + kw5).astype(f32)
    w1c = jnp.einsum('tqh,isjw,hwc->tiqsjc', sel1, sel2, wc1k)
    w1c = w1c.reshape(6, 28, 2, 2, 120)
    w1c = jnp.pad(w1c, ((0, 0), (0, 0), (0, 0), (0, 0), (0, 8)))
    w1c = w1c.reshape(168, 512).astype(bf16)

    wc2k = w2.reshape(25, 128, 128)[:, :10, :20].reshape(5, 5, 10, 20)
    j1a = jnp.arange(12)[:, None, None, None]
    s2b = jnp.arange(2)[None, :, None, None]
    j2a = jnp.arange(4)[None, None, :, None]
    kw5b = jnp.arange(5)[None, None, None, :]
    sel3 = (j1a == 2 * j2a + s2b + kw5b).astype(f32)
    w2c = jnp.einsum('jszw,hwio->hjiszo', sel3, wc2k)
    w2c = w2c.reshape(5, 120, 2, 80)
    w2c = jnp.pad(w2c, ((0, 0), (0, 8), (0, 0), (0, 48)))
    w2c = w2c.reshape(640, 256).astype(bf16)

    wf1c = wf1p.reshape(4, 4, 128, 128)[:, :, :20, :].reshape(4, 80, 128)
    wf1c = jnp.pad(wf1c, ((0, 0), (0, 48), (0, 0))).reshape(512, 128)
    wf1c = wf1c.astype(bf16)
    wf2c = wf2p.astype(bf16)

    lane = jnp.arange(128)
    b1v = jnp.where(lane < 120, bcat[0][lane % 10], 0.0)
    b2v = jnp.where(lane < 80, bcat[1][lane % 20], 0.0)
    bpack = jnp.stack([b1v, b2v, bcat[2], bcat[3]], axis=0)

    xr = x.reshape(B, 28, 28)
    xq = jnp.stack([xr[:, 0::2, :], xr[:, 1::2, :]], 0)
    xq = jnp.transpose(xq, (0, 2, 1, 3))
    xq = jnp.pad(xq, ((0, 0), (0, 4), (0, 0), (0, 0))).astype(bf16)

    flops = 2 * B * (16 * 168 * 512 + 8 * 640 * 256 + 512 * 128 + 128 * 128)
    bytes_accessed = xq.size * 2 + B * 128 * 4 + 2 * (
        w1c.size + w2c.size + wf1c.size + wf2c.size) + bpack.size * 4

    out = pl.pallas_call(
        _lenet_kernel,
        out_shape=jax.ShapeDtypeStruct((B, 128), f32),
        grid=(B // bt,),
        in_specs=[
            pl.BlockSpec((2, 18, bt, 28), lambda i: (0, 0, i, 0)),
            _const_spec((168, 512)),
            _const_spec((640, 256)),
            _const_spec((512, 128)),
            _const_spec((128, 128)),
            _const_spec((4, 128)),
        ],
        out_specs=pl.BlockSpec((bt, 128), lambda i: (i, 0)),
        compiler_params=pltpu.CompilerParams(
            dimension_semantics=("parallel",),
            vmem_limit_bytes=64 * 1024 * 1024),
        cost_estimate=pl.CostEstimate(flops=flops, transcendentals=B * 128,
                                      bytes_accessed=bytes_accessed),
    )(xq, w1c, w2c, wf1c, wf2c, bpack)
    return out[:, :10]
